# baseline (speedup 1.0000x reference)
import jax
import jax.numpy as jnp
from jax import lax
from jax.experimental import pallas as pl
from jax.experimental.pallas import tpu as pltpu
from jax.experimental.pallas import tpu_sc as plsc

_N = 1000
_F = 26
_NCOLS = _F * _N
_ROWBUF = _NCOLS + 16
_NW = 32
_NBUF = 4


def _make_sc_kernel(b_per_w):
    nbuf = min(_NBUF, b_per_w)
    mesh = plsc.VectorSubcoreMesh(core_axis_name="c", subcore_axis_name="s")

    def body(sh_hbm, out_hbm, idx_v, bufs, sems):
        wid = lax.axis_index("s") * 2 + lax.axis_index("c")
        base = wid * b_per_w
        pltpu.sync_copy(sh_hbm.at[pl.ds(base, b_per_w)], idx_v)

        ones = jnp.full((16,), 1.0, jnp.float32)
        zeros = jnp.zeros((16,), jnp.float32)

        def zero_buf(k, buf):
            @pl.loop(0, _ROWBUF // 16)
            def _(i):
                buf[pl.ds(i * 16, 16)] = zeros

        for k in range(nbuf):
            zero_buf(k, bufs[k])

        def prep(buf, row):
            i0 = idx_v[row, pl.ds(0, 16)]
            i1 = idx_v[row, pl.ds(16, 16)]
            plsc.store_scatter(buf, [i0], ones)
            plsc.store_scatter(buf, [i1], ones)

        def reset(buf, row):
            i0 = idx_v[row, pl.ds(0, 16)]
            i1 = idx_v[row, pl.ds(16, 16)]
            plsc.store_scatter(buf, [i0], zeros)
            plsc.store_scatter(buf, [i1], zeros)

        def fire(buf, sem, row):
            return pltpu.async_copy(
                buf.at[pl.ds(0, _NCOLS)], out_hbm.at[base + row], sem)

        def wait(buf, sem, row):
            pltpu.make_async_copy(
                buf.at[pl.ds(0, _NCOLS)], out_hbm.at[base + row], sem).wait()

        for b in range(nbuf):
            prep(bufs[b], b)
            fire(bufs[b], sems[b], b)

        @pl.loop(0, b_per_w // nbuf - 1)
        def _(it):
            done0 = it * nbuf
            for b in range(nbuf):
                wait(bufs[b], sems[b], done0 + b)
                reset(bufs[b], done0 + b)
                prep(bufs[b], done0 + nbuf + b)
                fire(bufs[b], sems[b], done0 + nbuf + b)

        last0 = b_per_w - nbuf
        for b in range(nbuf):
            wait(bufs[b], sems[b], last0 + b)

    return pl.kernel(
        body,
        out_type=jax.ShapeDtypeStruct((b_per_w * _NW, _NCOLS), jnp.float32),
        mesh=mesh,
        scratch_types=[
            pltpu.VMEM((b_per_w, 32), jnp.int32),
            [pltpu.VMEM((_ROWBUF,), jnp.float32) for _ in range(nbuf)],
            [pltpu.SemaphoreType.DMA for _ in range(nbuf)],
        ],
        compiler_params=pltpu.CompilerParams(
            needs_layout_passes=False,
            use_tc_tiling_on_sc=False,
        ),
    )


def kernel(x):
    if x.ndim == 1:
        x = x[None, :]
    b, f = x.shape
    assert f == _F

    shifted = x + (_N * jnp.arange(f, dtype=x.dtype))[None, :]
    sh = jnp.full((b, 32), _NCOLS, jnp.int32).at[:, :f].set(shifted)

    bp = -(-b // _NW) * _NW
    if bp != b:
        sh = jnp.pad(sh, ((0, bp - b), (0, 0)), constant_values=_NCOLS)

    out = _make_sc_kernel(bp // _NW)(sh)
    return out[:b]
